# baseline (speedup 1.0000x reference)
import functools

import jax
import jax.numpy as jnp
from jax import lax
from jax.experimental import pallas as pl
from jax.experimental.pallas import tpu as pltpu
from jax.experimental.pallas import tpu_sc as plsc

B = 16384
D = 32
NC = 2
NS = 16
NW = NC * NS
BPW = B // NW
CHUNK = 128
NCHUNK = BPW // CHUNK


def _sc_body(users_h, movies_h, ut_h, mt_h, wb_h, out_h,
             uidx, midx, urows, mrows, wbv, outv, sem):
    wid = lax.axis_index("s") * NC + lax.axis_index("c")
    base = wid * BPW

    pltpu.sync_copy(users_h.at[pl.ds(base, BPW)], uidx)
    pltpu.sync_copy(movies_h.at[pl.ds(base, BPW)], midx)
    pltpu.sync_copy(wb_h, wbv)

    copies = []
    for c in range(NCHUNK):
        sl = pl.ds(c * CHUNK, CHUNK)
        copies.append(pltpu.async_copy(ut_h.at[uidx.at[sl]], urows.at[sl], sem))
        copies.append(pltpu.async_copy(mt_h.at[midx.at[sl]], mrows.at[sl], sem))
    for cp in copies:
        cp.wait()

    wvecs = [wbv[pl.ds(i * 16, 16)] for i in range(5)]
    wu = [wvecs[k // 16][k % 16] for k in range(D)]
    wm = [wvecs[(D + k) // 16][(D + k) % 16] for k in range(D)]
    bias = wvecs[4][0]
    iota = lax.iota(jnp.int32, 16)
    cols = [jnp.full((16,), k, jnp.int32) for k in range(D)]

    def group(j, carry):
        row = iota + j * 16
        acc = jnp.full((16,), bias, jnp.float32)
        for k in range(D):
            acc = acc + plsc.load_gather(urows, [row, cols[k]]) * wu[k]
            acc = acc + plsc.load_gather(mrows, [row, cols[k]]) * wm[k]
        outv[pl.ds(j * 16, 16)] = acc
        return carry

    lax.fori_loop(0, BPW // 16, group, 0)
    pltpu.sync_copy(outv, out_h.at[pl.ds(base, BPW)])


@functools.partial(jax.jit, static_argnames=())
def kernel(users, movies, user_table, movie_table, W, b):
    wb = jnp.concatenate(
        [W.reshape(-1), b.reshape(-1), jnp.zeros((15,), jnp.float32)])
    run = pl.kernel(
        _sc_body,
        mesh=plsc.VectorSubcoreMesh(core_axis_name="c", subcore_axis_name="s"),
        compiler_params=pltpu.CompilerParams(
            needs_layout_passes=False, use_tc_tiling_on_sc=False),
        out_type=jax.ShapeDtypeStruct((B,), jnp.float32),
        scratch_types=[
            pltpu.VMEM((BPW,), jnp.int32),
            pltpu.VMEM((BPW,), jnp.int32),
            pltpu.VMEM((BPW, D), jnp.float32),
            pltpu.VMEM((BPW, D), jnp.float32),
            pltpu.VMEM((2 * D + 16,), jnp.float32),
            pltpu.VMEM((BPW,), jnp.float32),
            pltpu.SemaphoreType.DMA,
        ],
    )
    out = run(users.astype(jnp.int32), movies.astype(jnp.int32),
              user_table, movie_table, wb)
    return out.reshape(B, 1)
